# baseline (speedup 1.0000x reference)
import functools

import jax
import jax.numpy as jnp
from jax import lax
from jax.experimental import pallas as pl
from jax.experimental.pallas import tpu as pltpu
from jax.experimental.pallas import tpu_sc as plsc

N = 10000
E = 320000
D = 128

NC = 2
NS = 16
NW = NC * NS
L = 16

NPAD = 10240
EPW = E // NW
K = 96
NCHUNK = 112
HCHUNK = NCHUNK // 2
EPT = NCHUNK * K
ACCR = 10112
SROWS = ACCR // NS

RB = 1000
GRID = N // RB

_mesh = plsc.VectorSubcoreMesh(
    core_axis_name="c", subcore_axis_name="s", num_cores=NC, num_subcores=NS)
_sc_params = pltpu.CompilerParams(needs_layout_passes=False)


@functools.partial(
    pl.kernel,
    out_type=jax.ShapeDtypeStruct((NW, NPAD), jnp.float32),
    mesh=_mesh,
    scratch_types=[
        pltpu.VMEM((EPW,), jnp.int32),
        pltpu.VMEM((NPAD,), jnp.float32),
    ],
    compiler_params=_sc_params,
)
def _sc_deg(dst_hbm, out_hbm, dst_v, deg_v):
    wid = lax.axis_index("c") * NS + lax.axis_index("s")
    pltpu.sync_copy(dst_hbm.at[wid], dst_v)

    zeros = jnp.zeros((L,), jnp.float32)

    def _zero(i, carry):
        deg_v[pl.ds(i * L, L)] = zeros
        return carry

    lax.fori_loop(0, NPAD // L, _zero, 0)

    ones = jnp.ones((L,), jnp.float32)

    def _count(i, carry):
        idx = dst_v[pl.ds(i * L, L)]
        plsc.addupdate_scatter(deg_v, [idx], ones)
        return carry

    lax.fori_loop(0, EPW // L, _count, 0)
    pltpu.sync_copy(deg_v, out_hbm.at[wid])


def _tc_g_body(x_ref, w_ref, degp_ref, g_ref):
    deg = jnp.sum(degp_ref[...], axis=1) + 1.0
    dis = lax.rsqrt(deg)
    h = lax.dot_general(
        x_ref[...], w_ref[...], (((1,), (1,)), ((), ())),
        preferred_element_type=jnp.float32,
        precision=lax.Precision.HIGHEST)
    g_ref[...] = h * dis[:, None]


def _tc_g(x, W, degp):
    return pl.pallas_call(
        _tc_g_body,
        grid=(GRID,),
        in_specs=[
            pl.BlockSpec((RB, D), lambda i: (i, 0)),
            pl.BlockSpec((D, D), lambda i: (0, 0)),
            pl.BlockSpec((RB, NW), lambda i: (i, 0)),
        ],
        out_specs=pl.BlockSpec((RB, D), lambda i: (i, 0)),
        out_shape=jax.ShapeDtypeStruct((N, D), jnp.float32),
    )(x, W, degp)


@functools.partial(
    pl.kernel,
    out_type=jax.ShapeDtypeStruct((NC, ACCR, D), jnp.float32),
    mesh=_mesh,
    scratch_types=[
        pltpu.VMEM((HCHUNK, K), jnp.int32),
        pltpu.VMEM((HCHUNK, K), jnp.int32),
        pltpu.VMEM((K, D), jnp.float32),
        pltpu.VMEM((K, D), jnp.float32),
        pltpu.SemaphoreType.DMA,
        pltpu.SemaphoreType.DMA,
        pltpu.VMEM_SHARED((ACCR, D), jnp.float32),
    ],
    compiler_params=_sc_params,
)
def _sc_scatter(g_hbm, src_hbm, dst_hbm, zrows_hbm, out_hbm,
                src_v, dst_v, buf0, buf1, sem0, sem1, acc):
    cid = lax.axis_index("c")
    sid = lax.axis_index("s")
    wid = cid * NS + sid

    pltpu.sync_copy(zrows_hbm, acc.at[pl.ds(sid * SROWS, SROWS)])
    plsc.subcore_barrier()

    for p in range(2):
        pltpu.sync_copy(src_hbm.at[wid, pl.ds(p * HCHUNK, HCHUNK)], src_v)
        pltpu.sync_copy(dst_hbm.at[wid, pl.ds(p * HCHUNK, HCHUNK)], dst_v)
        pltpu.async_copy(g_hbm.at[src_v.at[0]], buf0, sem0)

        def _pair(i, carry):
            c0 = 2 * i
            c1 = c0 + 1
            pltpu.async_copy(g_hbm.at[src_v.at[c1]], buf1, sem1)
            pltpu.make_async_copy(g_hbm.at[src_v.at[c0]], buf0, sem0).wait()
            pltpu.sync_copy(buf0, acc.at[dst_v.at[c0]], add=True)

            @pl.when(c1 + 1 < HCHUNK)
            def _():
                pltpu.async_copy(g_hbm.at[src_v.at[c1 + 1]], buf0, sem0)

            pltpu.make_async_copy(g_hbm.at[src_v.at[c1]], buf1, sem1).wait()
            pltpu.sync_copy(buf1, acc.at[dst_v.at[c1]], add=True)
            return carry

        lax.fori_loop(0, HCHUNK // 2, _pair, 0)

    plsc.subcore_barrier()
    pltpu.sync_copy(acc.at[pl.ds(sid * SROWS, SROWS)],
                    out_hbm.at[cid, pl.ds(sid * SROWS, SROWS)])


def _ln(h, gamma, beta):
    mu = jnp.mean(h, axis=-1, keepdims=True)
    var = jnp.mean((h - mu) ** 2, axis=-1, keepdims=True)
    return (h - mu) * lax.rsqrt(var + 1e-5) * gamma + beta


def _tc_epi_body(x_ref, g_ref, degp_ref, accp_ref, b_ref,
                 g1_ref, b1_ref, g2_ref, b2_ref, out_ref):
    deg = jnp.sum(degp_ref[...], axis=1) + 1.0
    dis = lax.rsqrt(deg)
    acc = accp_ref[0] + accp_ref[1]
    t = (acc + g_ref[...]) * dis[:, None] + b_ref[...]
    t = jnp.maximum(t, 0.0)
    t = _ln(t, g1_ref[...], b1_ref[...])
    t = x_ref[...] + t
    out_ref[...] = _ln(t, g2_ref[...], b2_ref[...])


def _tc_epilogue(x, g, degp, accp, b, g1, b1, g2, b2):
    vec = pl.BlockSpec((1, D), lambda i: (0, 0))
    return pl.pallas_call(
        _tc_epi_body,
        grid=(GRID,),
        in_specs=[
            pl.BlockSpec((RB, D), lambda i: (i, 0)),
            pl.BlockSpec((RB, D), lambda i: (i, 0)),
            pl.BlockSpec((RB, NW), lambda i: (i, 0)),
            pl.BlockSpec((NC, RB, D), lambda i: (0, i, 0)),
            vec, vec, vec, vec, vec,
        ],
        out_specs=pl.BlockSpec((RB, D), lambda i: (i, 0)),
        out_shape=jax.ShapeDtypeStruct((N, D), jnp.float32),
    )(x, g, degp, accp, b.reshape(1, D), g1.reshape(1, D),
      b1.reshape(1, D), g2.reshape(1, D), b2.reshape(1, D))


def kernel(x, edge_index, W, b, gamma1, beta1, gamma2, beta2):
    src = edge_index[0].reshape(NW, EPW)
    dst = edge_index[1].reshape(NW, EPW)
    pad = EPT - EPW
    src_p = jnp.pad(src, ((0, 0), (0, pad))).reshape(NW, NCHUNK, K)
    dst_p = jnp.pad(dst, ((0, 0), (0, pad)), constant_values=N)

    degp = _sc_deg(dst)
    degp_t = degp.T[:N]
    g = _tc_g(x, W, degp_t)
    zrows = jnp.zeros((SROWS, D), jnp.float32)
    accp = _sc_scatter(g, src_p, dst_p.reshape(NW, NCHUNK, K), zrows)
    return _tc_epilogue(x, g, degp_t, accp, b, gamma1, beta1, gamma2, beta2)


# speedup vs baseline: 2.9168x; 2.9168x over previous
import functools

import jax
import jax.numpy as jnp
from jax import lax
from jax.experimental import pallas as pl
from jax.experimental.pallas import tpu as pltpu
from jax.experimental.pallas import tpu_sc as plsc

N = 10000
E = 320000
D = 128

NC = 2
NS = 16
NW = NC * NS
L = 16

NPAD = 10240
EPW = E // NW
K = 128
NCHUNK = 79
EPT = NCHUNK * K
ACCR = 10240
SROWS = ACCR // NS

RB = 1000
GRID = N // RB

_mesh = plsc.VectorSubcoreMesh(
    core_axis_name="c", subcore_axis_name="s", num_cores=NC, num_subcores=NS)
_sc_params = pltpu.CompilerParams(needs_layout_passes=False)


@functools.partial(
    pl.kernel,
    out_type=jax.ShapeDtypeStruct((NW, NPAD), jnp.float32),
    mesh=_mesh,
    scratch_types=[
        pltpu.VMEM((EPT,), jnp.int32),
        pltpu.VMEM((NPAD,), jnp.float32),
    ],
    compiler_params=_sc_params,
)
def _sc_deg(dst_hbm, out_hbm, dst_v, deg_v):
    wid = lax.axis_index("c") * NS + lax.axis_index("s")
    pltpu.sync_copy(dst_hbm.at[wid], dst_v)

    zeros = jnp.zeros((L,), jnp.float32)

    def _zero(i, carry):
        deg_v[pl.ds(i * L, L)] = zeros
        return carry

    lax.fori_loop(0, NPAD // L, _zero, 0)

    ones = jnp.ones((L,), jnp.float32)

    def _count(i, carry):
        idx = dst_v[pl.ds(i * L, L)]
        plsc.addupdate_scatter(deg_v, [idx], ones)
        return carry

    lax.fori_loop(0, EPT // L, _count, 0)
    pltpu.sync_copy(deg_v, out_hbm.at[wid])


def _tc_g_body(x_ref, w_ref, degp_ref, g_ref):
    deg = jnp.sum(degp_ref[...], axis=1) + 1.0
    dis = lax.rsqrt(deg)
    h = lax.dot_general(
        x_ref[...], w_ref[...], (((1,), (1,)), ((), ())),
        preferred_element_type=jnp.float32,
        precision=lax.Precision.HIGHEST)
    g_ref[...] = h * dis[:, None]


def _tc_g(x, W, degp):
    return pl.pallas_call(
        _tc_g_body,
        grid=(GRID,),
        in_specs=[
            pl.BlockSpec((RB, D), lambda i: (i, 0)),
            pl.BlockSpec((D, D), lambda i: (0, 0)),
            pl.BlockSpec((RB, NW), lambda i: (i, 0)),
        ],
        out_specs=pl.BlockSpec((RB, D), lambda i: (i, 0)),
        out_shape=jax.ShapeDtypeStruct((N, D), jnp.float32),
    )(x, W, degp)


@functools.partial(
    pl.kernel,
    out_type=jax.ShapeDtypeStruct((NC, ACCR, D), jnp.float32),
    mesh=_mesh,
    scratch_types=[
        pltpu.VMEM((NCHUNK, K), jnp.int32),
        pltpu.VMEM((NCHUNK, K), jnp.int32),
        pltpu.VMEM((K, D), jnp.float32),
        pltpu.SemaphoreType.DMA,
        pltpu.VMEM_SHARED((ACCR, D), jnp.float32),
    ],
    compiler_params=_sc_params,
)
def _sc_scatter(g_hbm, src_hbm, dst_hbm, out_hbm,
                src_v, dst_v, buf, sem, acc):
    cid = lax.axis_index("c")
    sid = lax.axis_index("s")
    wid = cid * NS + sid

    pltpu.sync_copy(src_hbm.at[wid], src_v)
    pltpu.sync_copy(dst_hbm.at[wid], dst_v)

    zeros = jnp.zeros((L,), jnp.float32)

    def _zrow(r, carry):
        for j in range(D // L):
            buf[r, pl.ds(j * L, L)] = zeros
        return carry

    lax.fori_loop(0, K, _zrow, 0)
    for z in range(SROWS // K):
        pltpu.sync_copy(buf, acc.at[pl.ds(sid * SROWS + z * K, K)])
    plsc.subcore_barrier()

    def _chunk(ci, carry):
        pltpu.async_copy(g_hbm.at[src_v.at[ci]], buf, sem).wait()
        pltpu.sync_copy(buf, acc.at[dst_v.at[ci]], add=True)
        return carry

    lax.fori_loop(0, NCHUNK, _chunk, 0)

    plsc.subcore_barrier()
    pltpu.sync_copy(acc.at[pl.ds(sid * SROWS, SROWS)],
                    out_hbm.at[cid, pl.ds(sid * SROWS, SROWS)])


def _ln(h, gamma, beta):
    mu = jnp.mean(h, axis=-1, keepdims=True)
    var = jnp.mean((h - mu) ** 2, axis=-1, keepdims=True)
    return (h - mu) * lax.rsqrt(var + 1e-5) * gamma + beta


def _tc_epi_body(x_ref, g_ref, degp_ref, accp_ref, b_ref,
                 g1_ref, b1_ref, g2_ref, b2_ref, out_ref):
    deg = jnp.sum(degp_ref[...], axis=1) + 1.0
    dis = lax.rsqrt(deg)
    acc = accp_ref[0] + accp_ref[1]
    t = (acc + g_ref[...]) * dis[:, None] + b_ref[...]
    t = jnp.maximum(t, 0.0)
    t = _ln(t, g1_ref[...], b1_ref[...])
    t = x_ref[...] + t
    out_ref[...] = _ln(t, g2_ref[...], b2_ref[...])


def _tc_epilogue(x, g, degp, accp, b, g1, b1, g2, b2):
    vec = pl.BlockSpec((1, D), lambda i: (0, 0))
    return pl.pallas_call(
        _tc_epi_body,
        grid=(GRID,),
        in_specs=[
            pl.BlockSpec((RB, D), lambda i: (i, 0)),
            pl.BlockSpec((RB, D), lambda i: (i, 0)),
            pl.BlockSpec((RB, NW), lambda i: (i, 0)),
            pl.BlockSpec((NC, RB, D), lambda i: (0, i, 0)),
            vec, vec, vec, vec, vec,
        ],
        out_specs=pl.BlockSpec((RB, D), lambda i: (i, 0)),
        out_shape=jax.ShapeDtypeStruct((N, D), jnp.float32),
    )(x, g, degp, accp, b.reshape(1, D), g1.reshape(1, D),
      b1.reshape(1, D), g2.reshape(1, D), b2.reshape(1, D))


def kernel(x, edge_index, W, b, gamma1, beta1, gamma2, beta2):
    src = edge_index[0].reshape(NW, EPW)
    dst = edge_index[1].reshape(NW, EPW)
    pad = EPT - EPW
    src_p = jnp.pad(src, ((0, 0), (0, pad))).reshape(NW, NCHUNK, K)
    dst_p = jnp.pad(dst, ((0, 0), (0, pad)), constant_values=N)

    degp = _sc_deg(dst_p)
    degp_t = degp.T[:N]
    g = _tc_g(x, W, degp_t)
    accp = _sc_scatter(g, src_p, dst_p.reshape(NW, NCHUNK, K))
    return _tc_epilogue(x, g, degp_t, accp, b, gamma1, beta1, gamma2, beta2)
